# baseline (speedup 1.0000x reference)
import math

import jax
import jax.numpy as jnp
from jax import lax
from jax.experimental import pallas as pl

B, N1, N2 = 4, 512, 2048
IN_C, OUT_C = 384, 512
RADIUS, NSAMPLE = 0.3, 32
FEAT_DIM = IN_C // 6
EPS = 1e-5
ALPHA0 = 100.0 / RADIUS
NEG_LOG1000_OVER_FD = -math.log(1000.0) / FEAT_DIM


def _pospool_body(qx_ref, sxT_ref, feat_ref, wT_ref, smask_ref, tri_ref,
                  y_ref):
    f32 = jnp.float32
    bf16 = jnp.bfloat16
    qx = qx_ref[...]
    sxT = sxT_ref[...]

    qs = lax.dot_general(qx, sxT, (((1,), (0,)), ((), ())),
                         precision=lax.Precision.HIGHEST,
                         preferred_element_type=f32)
    q2 = jnp.sum(qx * qx, axis=1, keepdims=True)
    s2 = jnp.sum(sxT * sxT, axis=0, keepdims=True)
    d2 = q2 + s2 - 2.0 * qs
    valid = (d2 < RADIUS * RADIUS) & (smask_ref[...] > 0.0)
    v = jnp.where(valid, 1.0, 0.0).astype(bf16)

    inc = lax.dot_general(v, tri_ref[...], (((1,), (0,)), ((), ())),
                          preferred_element_type=f32)
    m = v * jnp.where(inc <= float(NSAMPLE), 1.0, 0.0).astype(bf16)
    cnt = jnp.minimum(inc[:, N2 - 1:N2], float(NSAMPLE))

    rowi = lax.broadcasted_iota(jnp.int32, (3 * FEAT_DIM, N2), 0)
    jrow = lax.rem(rowi, FEAT_DIM).astype(f32)
    alpha = ALPHA0 * jnp.exp(jrow * NEG_LOG1000_OVER_FD)
    srep = jnp.concatenate(
        [jnp.broadcast_to(sxT[d:d + 1, :], (FEAT_DIM, N2)) for d in range(3)],
        axis=0)
    phs = alpha * srep
    ssin, scos = jnp.sin(phs), jnp.cos(phs)

    def dup_rows(t):
        return jnp.concatenate(
            [t[0:FEAT_DIM], t[0:FEAT_DIM],
             t[FEAT_DIM:2 * FEAT_DIM], t[FEAT_DIM:2 * FEAT_DIM],
             t[2 * FEAT_DIM:], t[2 * FEAT_DIM:]], axis=0)

    feat = feat_ref[...]
    ta = feat * dup_rows(ssin).astype(bf16)
    tb = feat * dup_rows(scos).astype(bf16)

    dn = (((1,), (1,)), ((), ()))
    pa = lax.dot_general(m, ta, dn, preferred_element_type=f32)
    pb = lax.dot_general(m, tb, dn, preferred_element_type=f32)

    lanei = lax.broadcasted_iota(jnp.int32, (N1, 3 * FEAT_DIM), 1)
    jlane = lax.rem(lanei, FEAT_DIM).astype(f32)
    alphaq = ALPHA0 * jnp.exp(jlane * NEG_LOG1000_OVER_FD)
    qrep = jnp.concatenate(
        [jnp.broadcast_to(qx[:, d:d + 1], (N1, FEAT_DIM)) for d in range(3)],
        axis=1)
    phq = alphaq * qrep
    qs, qc = jnp.sin(phq), jnp.cos(phq)

    def dup_lanes(t):
        return jnp.concatenate(
            [t[:, 0:FEAT_DIM], t[:, 0:FEAT_DIM],
             t[:, FEAT_DIM:2 * FEAT_DIM], t[:, FEAT_DIM:2 * FEAT_DIM],
             t[:, 2 * FEAT_DIM:], t[:, 2 * FEAT_DIM:]], axis=1)

    qs4, qc4 = dup_lanes(qs), dup_lanes(qc)
    r = lax.rem(lax.broadcasted_iota(jnp.int32, (N1, IN_C), 1), 2 * FEAT_DIM)
    is_sin = r < FEAT_DIM
    x = jnp.where(is_sin, pa, pb)
    y = jnp.where(is_sin, -pb, pa)
    ofeat = (qc4 * x + qs4 * y) / cnt

    y_ref[...] = jnp.dot(ofeat, wT_ref[...], preferred_element_type=f32)


def _bn_body(y_ref, g_ref, b_ref, out_ref):
    y = y_ref[...]
    mean = jnp.mean(y, axis=0, keepdims=True)
    d = y - mean
    var = jnp.mean(d * d, axis=0, keepdims=True)
    o = d * lax.rsqrt(var + EPS) * g_ref[...] + b_ref[...]
    o = jnp.maximum(o, 0.0)
    for b in range(B):
        out_ref[b] = o[b * N1:(b + 1) * N1, :].T


def kernel(query_xyz, support_xyz, query_mask, support_mask,
           support_features, conv_w, bn_gamma, bn_beta):
    sxT = jnp.transpose(support_xyz, (0, 2, 1))
    wT = jnp.transpose(conv_w)
    feat_bf = support_features.astype(jnp.bfloat16)
    tri = jnp.triu(jnp.ones((N2, N2), jnp.bfloat16))
    y = pl.pallas_call(
        _pospool_body,
        grid=(B,),
        in_specs=[
            pl.BlockSpec((None, N1, 3), lambda b: (b, 0, 0)),
            pl.BlockSpec((None, 3, N2), lambda b: (b, 0, 0)),
            pl.BlockSpec((None, IN_C, N2), lambda b: (b, 0, 0)),
            pl.BlockSpec((IN_C, OUT_C), lambda b: (0, 0)),
            pl.BlockSpec((None, 1, N2), lambda b: (b, 0, 0)),
            pl.BlockSpec((N2, N2), lambda b: (0, 0)),
        ],
        out_specs=pl.BlockSpec((None, N1, OUT_C), lambda b: (b, 0, 0)),
        out_shape=jax.ShapeDtypeStruct((B, N1, OUT_C), jnp.float32),
    )(query_xyz, sxT, feat_bf, wT, support_mask[:, None, :], tri)

    out = pl.pallas_call(
        _bn_body,
        out_shape=jax.ShapeDtypeStruct((B, OUT_C, N1), jnp.float32),
    )(y.reshape(B * N1, OUT_C), bn_gamma[None, :], bn_beta[None, :])
    return out


# speedup vs baseline: 1.7153x; 1.7153x over previous
import math

import jax
import jax.numpy as jnp
from jax import lax
from jax.experimental import pallas as pl
from jax.experimental.pallas import tpu as pltpu

B, N1, N2 = 4, 512, 2048
IN_C, OUT_C = 384, 512
RADIUS, NSAMPLE = 0.3, 32
FEAT_DIM = IN_C // 6
EPS = 1e-5
ALPHA0 = 100.0 / RADIUS
NEG_LOG1000_OVER_FD = -math.log(1000.0) / FEAT_DIM
NCH = 16
CW = N2 // NCH


def _fast_sincos(x):
    kf = jnp.floor(x * (1.0 / math.pi) + 0.5)
    r = x - kf * 3.14159274101257324
    r = r - kf * -8.74227765734758577e-08
    r2 = r * r
    sp = -2.3889859e-08
    sp = sp * r2 + 2.7525562e-06
    sp = sp * r2 + -1.9839334e-04
    sp = sp * r2 + 8.3333310e-03
    sp = sp * r2 + -1.6666667e-01
    sinr = r + r * r2 * sp
    cp = -2.6051615e-07
    cp = cp * r2 + 2.4760495e-05
    cp = cp * r2 + -1.3888885e-03
    cp = cp * r2 + 4.1666666e-02
    cp = cp * r2 + -5.0000000e-01
    cosr = 1.0 + r2 * cp
    parity = kf - 2.0 * jnp.floor(kf * 0.5)
    sign = 1.0 - 2.0 * parity
    return sinr * sign, cosr * sign


def _pospool_body(qx_ref, sxT_ref, feat_ref, wT_ref, smask_ref,
                  gamma_ref, beta_ref, out_ref, ysc_ref):
    f32 = jnp.float32
    bf16 = jnp.bfloat16
    b = pl.program_id(0)

    qx = qx_ref[...]
    sxT = sxT_ref[...]

    qdots = lax.dot_general(qx, sxT, (((1,), (0,)), ((), ())),
                            precision=lax.Precision.HIGHEST,
                            preferred_element_type=f32)
    q2 = jnp.sum(qx * qx, axis=1, keepdims=True)
    s2 = jnp.sum(sxT * sxT, axis=0, keepdims=True)
    d2 = q2 + s2 - 2.0 * qdots
    valid = (d2 < RADIUS * RADIUS) & (smask_ref[...] > 0.0)
    v = jnp.where(valid, 1.0, 0.0).astype(bf16)

    ki = lax.broadcasted_iota(jnp.int32, (N2, NCH), 0)
    ci = lax.broadcasted_iota(jnp.int32, (N2, NCH), 1)
    ebf = jnp.where(ki < ci * CW, 1.0, 0.0).astype(bf16)
    base = lax.dot_general(v, ebf, (((1,), (0,)), ((), ())),
                           preferred_element_type=f32)
    i2 = lax.broadcasted_iota(jnp.int32, (CW, CW), 0)
    j2 = lax.broadcasted_iota(jnp.int32, (CW, CW), 1)
    l128 = jnp.where(i2 <= j2, 1.0, 0.0).astype(bf16)
    mparts = []
    cnt = None
    for c in range(NCH):
        vc = v[:, c * CW:(c + 1) * CW]
        incc = lax.dot_general(vc, l128, (((1,), (0,)), ((), ())),
                               preferred_element_type=f32) + base[:, c:c + 1]
        mparts.append(vc * jnp.where(incc <= float(NSAMPLE), 1.0, 0.0
                                     ).astype(bf16))
        if c == NCH - 1:
            cnt = jnp.minimum(incc[:, CW - 1:CW], float(NSAMPLE))
    m = jnp.concatenate(mparts, axis=1)

    rowi = lax.broadcasted_iota(jnp.int32, (3 * FEAT_DIM, N2), 0)
    jrow = lax.rem(rowi, FEAT_DIM).astype(f32)
    alpha = ALPHA0 * jnp.exp(jrow * NEG_LOG1000_OVER_FD)
    srep = jnp.concatenate(
        [jnp.broadcast_to(sxT[d:d + 1, :], (FEAT_DIM, N2)) for d in range(3)],
        axis=0)
    phs = alpha * srep
    ssin, scos = _fast_sincos(phs)

    def dup_rows(t):
        return jnp.concatenate(
            [t[0:FEAT_DIM], t[0:FEAT_DIM],
             t[FEAT_DIM:2 * FEAT_DIM], t[FEAT_DIM:2 * FEAT_DIM],
             t[2 * FEAT_DIM:], t[2 * FEAT_DIM:]], axis=0)

    feat = feat_ref[...]
    ta = feat * dup_rows(ssin).astype(bf16)
    tb = feat * dup_rows(scos).astype(bf16)

    dn = (((1,), (1,)), ((), ()))
    pa = lax.dot_general(m, ta, dn, preferred_element_type=f32)
    pb = lax.dot_general(m, tb, dn, preferred_element_type=f32)

    lanei = lax.broadcasted_iota(jnp.int32, (N1, 3 * FEAT_DIM), 1)
    jlane = lax.rem(lanei, FEAT_DIM).astype(f32)
    alphaq = ALPHA0 * jnp.exp(jlane * NEG_LOG1000_OVER_FD)
    qrep = jnp.concatenate(
        [jnp.broadcast_to(qx[:, d:d + 1], (N1, FEAT_DIM)) for d in range(3)],
        axis=1)
    phq = alphaq * qrep
    qsn, qcs = _fast_sincos(phq)

    def dup_lanes(t):
        return jnp.concatenate(
            [t[:, 0:FEAT_DIM], t[:, 0:FEAT_DIM],
             t[:, FEAT_DIM:2 * FEAT_DIM], t[:, FEAT_DIM:2 * FEAT_DIM],
             t[:, 2 * FEAT_DIM:], t[:, 2 * FEAT_DIM:]], axis=1)

    qs4, qc4 = dup_lanes(qsn), dup_lanes(qcs)
    r = lax.rem(lax.broadcasted_iota(jnp.int32, (N1, IN_C), 1), 2 * FEAT_DIM)
    is_sin = r < FEAT_DIM
    x = jnp.where(is_sin, pa, pb)
    y = jnp.where(is_sin, -pb, pa)
    ofeat = (qc4 * x + qs4 * y) / cnt

    yb = jnp.dot(ofeat, wT_ref[...], preferred_element_type=f32)
    ysc_ref[pl.ds(b * N1, N1), :] = yb

    @pl.when(b == B - 1)
    def _bn():
        yall = ysc_ref[...]
        mean = jnp.mean(yall, axis=0, keepdims=True)
        dev = yall - mean
        var = jnp.mean(dev * dev, axis=0, keepdims=True)
        o = dev * lax.rsqrt(var + EPS) * gamma_ref[...] + beta_ref[...]
        o = jnp.maximum(o, 0.0)
        for bb in range(B):
            out_ref[bb] = o[bb * N1:(bb + 1) * N1, :].T


def kernel(query_xyz, support_xyz, query_mask, support_mask,
           support_features, conv_w, bn_gamma, bn_beta):
    sxT = jnp.transpose(support_xyz, (0, 2, 1))
    wT = jnp.transpose(conv_w)
    feat_bf = support_features.astype(jnp.bfloat16)
    out = pl.pallas_call(
        _pospool_body,
        grid=(B,),
        in_specs=[
            pl.BlockSpec((None, N1, 3), lambda b: (b, 0, 0)),
            pl.BlockSpec((None, 3, N2), lambda b: (b, 0, 0)),
            pl.BlockSpec((None, IN_C, N2), lambda b: (b, 0, 0)),
            pl.BlockSpec((IN_C, OUT_C), lambda b: (0, 0)),
            pl.BlockSpec((None, 1, N2), lambda b: (b, 0, 0)),
            pl.BlockSpec((1, OUT_C), lambda b: (0, 0)),
            pl.BlockSpec((1, OUT_C), lambda b: (0, 0)),
        ],
        out_specs=pl.BlockSpec((B, OUT_C, N1), lambda b: (0, 0, 0)),
        out_shape=jax.ShapeDtypeStruct((B, OUT_C, N1), jnp.float32),
        scratch_shapes=[pltpu.VMEM((B * N1, OUT_C), jnp.float32)],
    )(query_xyz, sxT, feat_bf, wT, support_mask[:, None, :],
      bn_gamma[None, :], bn_beta[None, :])
    return out


# speedup vs baseline: 2.0488x; 1.1944x over previous
import math

import jax
import jax.numpy as jnp
from jax import lax
from jax.experimental import pallas as pl
from jax.experimental.pallas import tpu as pltpu

B, N1, N2 = 4, 512, 2048
IN_C, OUT_C = 384, 512
RADIUS, NSAMPLE = 0.3, 32
FEAT_DIM = IN_C // 6
EPS = 1e-5
ALPHA0 = 100.0 / RADIUS
NEG_LOG1000_OVER_FD = -math.log(1000.0) / FEAT_DIM
NCH = 16
CW = N2 // NCH


def _fast_sincos(x):
    kf = jnp.floor(x * (1.0 / math.pi) + 0.5)
    r = x - kf * 3.14159274101257324
    r = r - kf * -8.74227765734758577e-08
    r2 = r * r
    sp = -2.3889859e-08
    sp = sp * r2 + 2.7525562e-06
    sp = sp * r2 + -1.9839334e-04
    sp = sp * r2 + 8.3333310e-03
    sp = sp * r2 + -1.6666667e-01
    sinr = r + r * r2 * sp
    cp = -2.6051615e-07
    cp = cp * r2 + 2.4760495e-05
    cp = cp * r2 + -1.3888885e-03
    cp = cp * r2 + 4.1666666e-02
    cp = cp * r2 + -5.0000000e-01
    cosr = 1.0 + r2 * cp
    parity = kf - 2.0 * jnp.floor(kf * 0.5)
    sign = 1.0 - 2.0 * parity
    return sinr * sign, cosr * sign


def _pospool_body(qx_ref, sxT_ref, feat_ref, wT_ref, smask_ref,
                  gamma_ref, beta_ref, out_ref, ysc_ref):
    f32 = jnp.float32
    bf16 = jnp.bfloat16
    b = pl.program_id(0)

    qx = qx_ref[...]
    sxT = sxT_ref[...]

    d2 = jnp.zeros((N1, N2), f32)
    for d in range(3):
        diff = qx[:, d:d + 1] - sxT[d:d + 1, :]
        d2 = d2 + diff * diff
    valid = (d2 < RADIUS * RADIUS) & (smask_ref[...] > 0.0)
    v = jnp.where(valid, 1.0, 0.0).astype(bf16)

    ki = lax.broadcasted_iota(jnp.int32, (N2, NCH), 0)
    ci = lax.broadcasted_iota(jnp.int32, (N2, NCH), 1)
    ebf = jnp.where(ki < ci * CW, 1.0, 0.0).astype(bf16)
    base = lax.dot_general(v, ebf, (((1,), (0,)), ((), ())),
                           preferred_element_type=f32)
    i2 = lax.broadcasted_iota(jnp.int32, (CW, CW), 0)
    j2 = lax.broadcasted_iota(jnp.int32, (CW, CW), 1)
    l128 = jnp.where(i2 <= j2, 1.0, 0.0).astype(bf16)
    mparts = []
    cnt = None
    for c in range(NCH):
        vc = v[:, c * CW:(c + 1) * CW]
        incc = lax.dot_general(vc, l128, (((1,), (0,)), ((), ())),
                               preferred_element_type=f32) + base[:, c:c + 1]
        mparts.append(vc * jnp.where(incc <= float(NSAMPLE), 1.0, 0.0
                                     ).astype(bf16))
        if c == NCH - 1:
            cnt = jnp.minimum(incc[:, CW - 1:CW], float(NSAMPLE))
    m = jnp.concatenate(mparts, axis=1)

    rowi = lax.broadcasted_iota(jnp.int32, (3 * FEAT_DIM, N2), 0)
    jrow = lax.rem(rowi, FEAT_DIM).astype(f32)
    alpha = ALPHA0 * jnp.exp(jrow * NEG_LOG1000_OVER_FD)
    srep = jnp.concatenate(
        [jnp.broadcast_to(sxT[d:d + 1, :], (FEAT_DIM, N2)) for d in range(3)],
        axis=0)
    phs = alpha * srep
    ssin, scos = _fast_sincos(phs)

    def dup_rows(t):
        return jnp.concatenate(
            [t[0:FEAT_DIM], t[0:FEAT_DIM],
             t[FEAT_DIM:2 * FEAT_DIM], t[FEAT_DIM:2 * FEAT_DIM],
             t[2 * FEAT_DIM:], t[2 * FEAT_DIM:]], axis=0)

    feat = feat_ref[...]
    ta = feat * dup_rows(ssin).astype(bf16)
    tb = feat * dup_rows(scos).astype(bf16)

    dn = (((1,), (1,)), ((), ()))
    pa = lax.dot_general(m, ta, dn, preferred_element_type=f32)
    pb = lax.dot_general(m, tb, dn, preferred_element_type=f32)

    lanei = lax.broadcasted_iota(jnp.int32, (N1, 3 * FEAT_DIM), 1)
    jlane = lax.rem(lanei, FEAT_DIM).astype(f32)
    alphaq = ALPHA0 * jnp.exp(jlane * NEG_LOG1000_OVER_FD)
    qrep = jnp.concatenate(
        [jnp.broadcast_to(qx[:, d:d + 1], (N1, FEAT_DIM)) for d in range(3)],
        axis=1)
    phq = alphaq * qrep
    qsn, qcs = _fast_sincos(phq)

    def dup_lanes(t):
        return jnp.concatenate(
            [t[:, 0:FEAT_DIM], t[:, 0:FEAT_DIM],
             t[:, FEAT_DIM:2 * FEAT_DIM], t[:, FEAT_DIM:2 * FEAT_DIM],
             t[:, 2 * FEAT_DIM:], t[:, 2 * FEAT_DIM:]], axis=1)

    qs4, qc4 = dup_lanes(qsn), dup_lanes(qcs)
    r = lax.rem(lax.broadcasted_iota(jnp.int32, (N1, IN_C), 1), 2 * FEAT_DIM)
    is_sin = r < FEAT_DIM
    x = jnp.where(is_sin, pa, pb)
    y = jnp.where(is_sin, -pb, pa)
    ofeat = (qc4 * x + qs4 * y) / cnt

    yb = jnp.dot(ofeat, wT_ref[...], preferred_element_type=f32)
    ysc_ref[pl.ds(b * N1, N1), :] = yb

    @pl.when(b == B - 1)
    def _bn():
        yall = ysc_ref[...]
        mean = jnp.mean(yall, axis=0, keepdims=True)
        dev = yall - mean
        var = jnp.mean(dev * dev, axis=0, keepdims=True)
        o = dev * lax.rsqrt(var + EPS) * gamma_ref[...] + beta_ref[...]
        o = jnp.maximum(o, 0.0)
        for bb in range(B):
            out_ref[bb] = o[bb * N1:(bb + 1) * N1, :].T


def kernel(query_xyz, support_xyz, query_mask, support_mask,
           support_features, conv_w, bn_gamma, bn_beta):
    sxT = jnp.transpose(support_xyz, (0, 2, 1))
    wT = jnp.transpose(conv_w)
    feat_bf = support_features.astype(jnp.bfloat16)
    out = pl.pallas_call(
        _pospool_body,
        grid=(B,),
        in_specs=[
            pl.BlockSpec((None, N1, 3), lambda b: (b, 0, 0)),
            pl.BlockSpec((None, 3, N2), lambda b: (b, 0, 0)),
            pl.BlockSpec((None, IN_C, N2), lambda b: (b, 0, 0)),
            pl.BlockSpec((IN_C, OUT_C), lambda b: (0, 0)),
            pl.BlockSpec((None, 1, N2), lambda b: (b, 0, 0)),
            pl.BlockSpec((1, OUT_C), lambda b: (0, 0)),
            pl.BlockSpec((1, OUT_C), lambda b: (0, 0)),
        ],
        out_specs=pl.BlockSpec((B, OUT_C, N1), lambda b: (0, 0, 0)),
        out_shape=jax.ShapeDtypeStruct((B, OUT_C, N1), jnp.float32),
        scratch_shapes=[pltpu.VMEM((B * N1, OUT_C), jnp.float32)],
    )(query_xyz, sxT, feat_bf, wT, support_mask[:, None, :],
      bn_gamma[None, :], bn_beta[None, :])
    return out


# speedup vs baseline: 2.0764x; 1.0135x over previous
import math

import jax
import jax.numpy as jnp
from jax import lax
from jax.experimental import pallas as pl
from jax.experimental.pallas import tpu as pltpu

B, N1, N2 = 4, 512, 2048
IN_C, OUT_C = 384, 512
RADIUS, NSAMPLE = 0.3, 32
FEAT_DIM = IN_C // 6
EPS = 1e-5
ALPHA0 = 100.0 / RADIUS
NEG_LOG1000_OVER_FD = -math.log(1000.0) / FEAT_DIM
NCH = 16
CW = N2 // NCH


def _fast_sincos(x):
    kf = jnp.floor(x * (1.0 / math.pi) + 0.5)
    r = x - kf * 3.14159274101257324
    r = r - kf * -8.74227765734758577e-08
    r2 = r * r
    sp = -2.3889859e-08
    sp = sp * r2 + 2.7525562e-06
    sp = sp * r2 + -1.9839334e-04
    sp = sp * r2 + 8.3333310e-03
    sp = sp * r2 + -1.6666667e-01
    sinr = r + r * r2 * sp
    cp = -2.6051615e-07
    cp = cp * r2 + 2.4760495e-05
    cp = cp * r2 + -1.3888885e-03
    cp = cp * r2 + 4.1666666e-02
    cp = cp * r2 + -5.0000000e-01
    cosr = 1.0 + r2 * cp
    parity = kf - 2.0 * jnp.floor(kf * 0.5)
    sign = 1.0 - 2.0 * parity
    return sinr * sign, cosr * sign


def _pospool_body(qx_ref, qxT_ref, sxT_ref, feat_ref, w_ref, smask_ref,
                  gamma_ref, beta_ref, out_ref, ysc_ref):
    f32 = jnp.float32
    bf16 = jnp.bfloat16
    b = pl.program_id(0)

    qx = qx_ref[...]
    qxT = qxT_ref[...]
    sxT = sxT_ref[...]

    d2 = jnp.zeros((N1, N2), f32)
    for d in range(3):
        diff = qx[:, d:d + 1] - sxT[d:d + 1, :]
        d2 = d2 + diff * diff
    valid = (d2 < RADIUS * RADIUS) & (smask_ref[...] > 0.0)
    v = jnp.where(valid, 1.0, 0.0).astype(bf16)

    ki = lax.broadcasted_iota(jnp.int32, (N2, NCH), 0)
    ci = lax.broadcasted_iota(jnp.int32, (N2, NCH), 1)
    ebf = jnp.where(ki < ci * CW, 1.0, 0.0).astype(bf16)
    base = lax.dot_general(v, ebf, (((1,), (0,)), ((), ())),
                           preferred_element_type=f32)
    i2 = lax.broadcasted_iota(jnp.int32, (CW, CW), 0)
    j2 = lax.broadcasted_iota(jnp.int32, (CW, CW), 1)
    l128 = jnp.where(i2 <= j2, 1.0, 0.0).astype(bf16)
    mparts = []
    cnt = None
    for c in range(NCH):
        vc = v[:, c * CW:(c + 1) * CW]
        incc = lax.dot_general(vc, l128, (((1,), (0,)), ((), ())),
                               preferred_element_type=f32) + base[:, c:c + 1]
        mparts.append(vc * jnp.where(incc <= float(NSAMPLE), 1.0, 0.0
                                     ).astype(bf16))
        if c == NCH - 1:
            cnt = jnp.minimum(incc[:, CW - 1:CW], float(NSAMPLE))
    m = jnp.concatenate(mparts, axis=1)

    rowi = lax.broadcasted_iota(jnp.int32, (3 * FEAT_DIM, N2), 0)
    jrow = lax.rem(rowi, FEAT_DIM).astype(f32)
    alpha = ALPHA0 * jnp.exp(jrow * NEG_LOG1000_OVER_FD)
    srep = jnp.concatenate(
        [jnp.broadcast_to(sxT[d:d + 1, :], (FEAT_DIM, N2)) for d in range(3)],
        axis=0)
    phs = alpha * srep
    ssin, scos = _fast_sincos(phs)

    def dup_rows(t):
        return jnp.concatenate(
            [t[0:FEAT_DIM], t[0:FEAT_DIM],
             t[FEAT_DIM:2 * FEAT_DIM], t[FEAT_DIM:2 * FEAT_DIM],
             t[2 * FEAT_DIM:], t[2 * FEAT_DIM:]], axis=0)

    feat = feat_ref[...]
    ta = feat * dup_rows(ssin).astype(bf16)
    tb = feat * dup_rows(scos).astype(bf16)

    dn = (((1,), (1,)), ((), ()))
    pa = lax.dot_general(ta, m, dn, preferred_element_type=f32)
    pb = lax.dot_general(tb, m, dn, preferred_element_type=f32)

    rqi = lax.broadcasted_iota(jnp.int32, (3 * FEAT_DIM, N1), 0)
    jq = lax.rem(rqi, FEAT_DIM).astype(f32)
    alphaq = ALPHA0 * jnp.exp(jq * NEG_LOG1000_OVER_FD)
    qrep = jnp.concatenate(
        [jnp.broadcast_to(qxT[d:d + 1, :], (FEAT_DIM, N1)) for d in range(3)],
        axis=0)
    phq = alphaq * qrep
    qsn, qcs = _fast_sincos(phq)

    qs4, qc4 = dup_rows(qsn), dup_rows(qcs)
    r = lax.rem(lax.broadcasted_iota(jnp.int32, (IN_C, N1), 0), 2 * FEAT_DIM)
    is_sin = r < FEAT_DIM
    x = jnp.where(is_sin, pa, pb)
    y = jnp.where(is_sin, -pb, pa)
    rcnt = (1.0 / cnt).T
    ofeat = (qc4 * x + qs4 * y) * rcnt

    ybT = jnp.dot(w_ref[...], ofeat, preferred_element_type=f32)
    ysc_ref[:, pl.ds(b * N1, N1)] = ybT

    @pl.when(b == B - 1)
    def _bn():
        yall = ysc_ref[...]
        mean = jnp.mean(yall, axis=1, keepdims=True)
        dev = yall - mean
        var = jnp.mean(dev * dev, axis=1, keepdims=True)
        o = dev * lax.rsqrt(var + EPS) * gamma_ref[...] + beta_ref[...]
        o = jnp.maximum(o, 0.0)
        for bb in range(B):
            out_ref[bb] = o[:, bb * N1:(bb + 1) * N1]


def kernel(query_xyz, support_xyz, query_mask, support_mask,
           support_features, conv_w, bn_gamma, bn_beta):
    qxT = jnp.transpose(query_xyz, (0, 2, 1))
    sxT = jnp.transpose(support_xyz, (0, 2, 1))
    feat_bf = support_features.astype(jnp.bfloat16)
    out = pl.pallas_call(
        _pospool_body,
        grid=(B,),
        in_specs=[
            pl.BlockSpec((None, N1, 3), lambda b: (b, 0, 0)),
            pl.BlockSpec((None, 3, N1), lambda b: (b, 0, 0)),
            pl.BlockSpec((None, 3, N2), lambda b: (b, 0, 0)),
            pl.BlockSpec((None, IN_C, N2), lambda b: (b, 0, 0)),
            pl.BlockSpec((OUT_C, IN_C), lambda b: (0, 0)),
            pl.BlockSpec((None, 1, N2), lambda b: (b, 0, 0)),
            pl.BlockSpec((OUT_C, 1), lambda b: (0, 0)),
            pl.BlockSpec((OUT_C, 1), lambda b: (0, 0)),
        ],
        out_specs=pl.BlockSpec((B, OUT_C, N1), lambda b: (0, 0, 0)),
        out_shape=jax.ShapeDtypeStruct((B, OUT_C, N1), jnp.float32),
        scratch_shapes=[pltpu.VMEM((OUT_C, B * N1), jnp.float32)],
    )(query_xyz, qxT, sxT, feat_bf, conv_w, support_mask[:, None, :],
      bn_gamma[:, None], bn_beta[:, None])
    return out
